# baseline (speedup 1.0000x reference)
import jax
import jax.numpy as jnp
from jax import lax
from jax.experimental import pallas as pl
from jax.experimental.pallas import tpu as pltpu
from jax.experimental.pallas import tpu_sc as plsc

NUM_CLASSES = 1000000
EMBED = 32
HIST = 50
BATCH = 16384
LANE_PAD = 128

NC = 2
NS = 16
NW = NC * NS

B_PER_W = BATCH // NW
CHUNK_B = 4
IDX_ROW = 112
IDX_VALID = 100
ROWS_PER_CHUNK = 2 * IDX_ROW
IDX_ROWS_PER_W = B_PER_W * HIST // IDX_VALID
N_CHUNKS = B_PER_W // CHUNK_B
OUT_PER_CHUNK = CHUNK_B * EMBED * LANE_PAD
NBUF = 3
N_ROUNDS = N_CHUNKS // NBUF + 1

TC_COLS = 8192
TC_SUB = TC_COLS // 4
TC_GRID = -(-NUM_CLASSES // TC_COLS)
N_PAD = TC_GRID * TC_COLS


def _tc_compact_kernel(w_ref, eye_ref, o_ref):
    eye = eye_ref[...]
    for j in range(4):
        blk = w_ref[:, TC_SUB * j:TC_SUB * (j + 1)]
        o_ref[:, 32 * j:32 * (j + 1)] = lax.dot_general(
            blk, eye, (((0,), (0,)), ((), ())),
            preferred_element_type=jnp.float32)


def _compact_table(Wt):
    return pl.pallas_call(
        _tc_compact_kernel,
        grid=(TC_GRID,),
        in_specs=[
            pl.BlockSpec((EMBED, TC_COLS), lambda i: (0, i)),
            pl.BlockSpec((EMBED, EMBED), lambda i: (0, 0)),
        ],
        out_specs=pl.BlockSpec((TC_SUB, 4 * EMBED), lambda i: (i, 0)),
        out_shape=jax.ShapeDtypeStruct(
            (N_PAD // 4, 4 * EMBED), jnp.float32),
    )(Wt, jnp.eye(EMBED, dtype=jnp.float32))


def _sc_kernel(x_hbm, w_hbm, out_hbm, idx_v, *rest):
    stagings = rest[0:NBUF]
    outbufs = rest[NBUF:2 * NBUF]
    sem_g = rest[2 * NBUF:3 * NBUF]
    sem_o = rest[3 * NBUF:4 * NBUF]

    wid = lax.axis_index("s") * NC + lax.axis_index("c")

    pltpu.sync_copy(x_hbm.at[pl.ds(wid * IDX_ROWS_PER_W, IDX_ROWS_PER_W)],
                    idx_v)

    def rbody(c, carry):
        for g in range(IDX_ROW // 16):
            v = idx_v[c, pl.ds(16 * g, 16)]
            rem = v & (TC_COLS - 1)
            idx_v[c, pl.ds(16 * g, 16)] = (
                (v - rem) + 4 * (rem & (TC_SUB - 1)) + (rem >> 11))
        return carry
    lax.fori_loop(0, IDX_ROWS_PER_W, rbody, 0)

    lane128 = lax.iota(jnp.int32, 16) * LANE_PAD
    out_base_w = wid * B_PER_W * EMBED * LANE_PAD

    def issue_gather(cc, b):
        pltpu.async_copy(w_hbm.at[idx_v.at[2 * cc]],
                         stagings[b].at[pl.ds(0, IDX_ROW)], sem_g[b])
        pltpu.async_copy(w_hbm.at[idx_v.at[2 * cc + 1]],
                         stagings[b].at[pl.ds(IDX_ROW, IDX_ROW)], sem_g[b])

    def wait_gather(b):
        pltpu.make_async_copy(w_hbm.at[pl.ds(0, ROWS_PER_CHUNK)],
                              stagings[b], sem_g[b]).wait()

    def out_copy(cc, b):
        base = out_base_w + cc * OUT_PER_CHUNK
        base = pl.multiple_of(base, 8)
        return pltpu.make_async_copy(
            outbufs[b], out_hbm.at[pl.ds(base, OUT_PER_CHUNK)], sem_o[b])

    for b in range(NBUF):
        issue_gather(b, b)

    def body(r, carry):
        for b in range(NBUF):
            cc = r * NBUF + b

            @pl.when(cc < N_CHUNKS)
            def _step():
                wait_gather(b)

                @pl.when(r > 0)
                def _wait_prev_out():
                    out_copy(cc - NBUF, b).wait()

                for s in range(CHUNK_B * HIST):
                    row = s + (s // IDX_VALID) * (IDX_ROW - IDX_VALID)
                    bb, l = divmod(s, HIST)
                    for h in range(2):
                        vals = stagings[b][row, pl.ds(16 * h, 16)]
                        idx = lane128 + ((bb * EMBED + 16 * h) * LANE_PAD + l)
                        plsc.store_scatter(outbufs[b], [idx], vals)

                out_copy(cc, b).start()

                @pl.when(cc + NBUF < N_CHUNKS)
                def _issue_next():
                    issue_gather(cc + NBUF, b)
        return carry

    lax.fori_loop(0, N_ROUNDS, body, 0)

    for b in range(NBUF):
        out_copy(N_CHUNKS - NBUF + b, b).wait()


@jax.jit
def kernel(x, W):
    x2 = x.reshape(NW * IDX_ROWS_PER_W, IDX_VALID).astype(jnp.int32)
    x2 = jnp.pad(x2, ((0, 0), (0, IDX_ROW - IDX_VALID)))
    w_rm = _compact_table(W.T).reshape(N_PAD, EMBED)
    mesh = plsc.VectorSubcoreMesh(core_axis_name="c", subcore_axis_name="s")
    scratch = (
        [pltpu.VMEM((IDX_ROWS_PER_W, IDX_ROW), jnp.int32)]
        + [pltpu.VMEM((ROWS_PER_CHUNK, EMBED), jnp.float32)] * NBUF
        + [pltpu.VMEM((OUT_PER_CHUNK,), jnp.float32)] * NBUF
        + [pltpu.SemaphoreType.DMA] * (2 * NBUF)
    )
    run = pl.kernel(
        _sc_kernel,
        out_type=jax.ShapeDtypeStruct((BATCH * EMBED * LANE_PAD,), jnp.float32),
        mesh=mesh,
        scratch_types=scratch,
        compiler_params=pltpu.CompilerParams(
            needs_layout_passes=False, use_tc_tiling_on_sc=False),
    )
    out = run(x2, w_rm)
    return out.reshape(BATCH, EMBED, LANE_PAD)[:, :, :HIST]


# speedup vs baseline: 2.8948x; 2.8948x over previous
import jax
import jax.numpy as jnp
from jax import lax
from jax.experimental import pallas as pl
from jax.experimental.pallas import tpu as pltpu
from jax.experimental.pallas import tpu_sc as plsc

NUM_CLASSES = 1000000
EMBED = 32
HIST = 50
BATCH = 16384

NC = 2
NS = 16
NW = NC * NS

B_PER_W = BATCH // NW
IDX_SUB = 128
N_SUB = B_PER_W // IDX_SUB
OB_PITCH = B_PER_W + 1
NBUF = 2
N_ROUNDS = HIST // NBUF

TC_COLS = 32768
TC_SUB = TC_COLS // 4
TC_GRID = -(-NUM_CLASSES // TC_COLS)
N_PAD = TC_GRID * TC_COLS


def _tc_compact_kernel(w_ref, o_ref):
    for j in range(4):
        o_ref[:, 32 * j:32 * (j + 1)] = w_ref[:, TC_SUB * j:TC_SUB * (j + 1)].T


def _compact_table(Wt):
    return pl.pallas_call(
        _tc_compact_kernel,
        grid=(TC_GRID,),
        in_specs=[pl.BlockSpec((EMBED, TC_COLS), lambda i: (0, i))],
        out_specs=pl.BlockSpec((TC_SUB, 4 * EMBED), lambda i: (i, 0)),
        out_shape=jax.ShapeDtypeStruct(
            (N_PAD // 4, 4 * EMBED), jnp.float32),
    )(Wt)


def _sc_kernel(x_hbm, w_hbm, out_hbm, idx_v, *rest):
    stagings = rest[0:NBUF]
    outbufs = rest[NBUF:2 * NBUF]
    sem_g = rest[2 * NBUF:3 * NBUF]
    sem_o = rest[3 * NBUF:4 * NBUF]

    wid = lax.axis_index("s") * NC + lax.axis_index("c")

    pltpu.sync_copy(x_hbm.at[:, wid], idx_v)

    def rbody(l, carry):
        for k in range(N_SUB):
            for g in range(IDX_SUB // 16):
                v = idx_v[l, k, pl.ds(16 * g, 16)]
                rem = v & (TC_COLS - 1)
                idx_v[l, k, pl.ds(16 * g, 16)] = (
                    (v - rem) + 4 * (rem & (TC_SUB - 1)) + (rem >> 13))
        return carry
    lax.fori_loop(0, HIST, rbody, 0)

    lane = lax.iota(jnp.int32, 16)
    rows_h = [lane + 16 * h for h in range(2)]

    def issue_gather(l, b):
        for k in range(N_SUB):
            pltpu.async_copy(w_hbm.at[idx_v.at[l, k]],
                             stagings[b].at[pl.ds(k * IDX_SUB, IDX_SUB)],
                             sem_g[b])

    def wait_gather(b):
        pltpu.make_async_copy(w_hbm.at[pl.ds(0, B_PER_W)],
                              stagings[b], sem_g[b]).wait()

    def out_copy(l, b):
        return pltpu.make_async_copy(
            outbufs[b].at[:, pl.ds(0, B_PER_W)],
            out_hbm.at[pl.ds(l * EMBED, EMBED), pl.ds(wid * B_PER_W, B_PER_W)],
            sem_o[b])

    for b in range(NBUF):
        issue_gather(b, b)

    def body(r, carry):
        for b in range(NBUF):
            l = r * NBUF + b
            wait_gather(b)

            @pl.when(r > 0)
            def _wait_prev_out():
                out_copy(l - NBUF, b).wait()

            for bp in range(B_PER_W):
                col = jnp.full((16,), bp, jnp.int32)
                for h in range(2):
                    vals = stagings[b][bp, pl.ds(16 * h, 16)]
                    plsc.store_scatter(outbufs[b], [rows_h[h], col], vals)

            out_copy(l, b).start()

            @pl.when(r < N_ROUNDS - 1)
            def _issue_next():
                issue_gather(l + NBUF, b)
        return carry

    lax.fori_loop(0, N_ROUNDS, body, 0)

    for b in range(NBUF):
        out_copy(HIST - NBUF + b, b).wait()


@jax.jit
def kernel(x, W):
    x4 = x.astype(jnp.int32).T.reshape(HIST, NW, N_SUB, IDX_SUB)
    w_rm = _compact_table(W.T).reshape(N_PAD, EMBED)
    mesh = plsc.VectorSubcoreMesh(core_axis_name="c", subcore_axis_name="s")
    scratch = (
        [pltpu.VMEM((HIST, N_SUB, IDX_SUB), jnp.int32)]
        + [pltpu.VMEM((B_PER_W, EMBED), jnp.float32)] * NBUF
        + [pltpu.VMEM((EMBED, OB_PITCH), jnp.float32)] * NBUF
        + [pltpu.SemaphoreType.DMA] * (2 * NBUF)
    )
    run = pl.kernel(
        _sc_kernel,
        out_type=jax.ShapeDtypeStruct((HIST * EMBED, BATCH), jnp.float32),
        mesh=mesh,
        scratch_types=scratch,
        compiler_params=pltpu.CompilerParams(
            needs_layout_passes=False, use_tc_tiling_on_sc=False),
    )
    out = run(x4, w_rm)
    return out.reshape(HIST, EMBED, BATCH).transpose(2, 1, 0)
